# Initial kernel scaffold; baseline (speedup 1.0000x reference)
import functools

import numpy as np
import jax
import jax.numpy as jnp
from jax import lax
from jax.experimental import pallas as pl
from jax.experimental.pallas import tpu as pltpu
from jax.experimental.pallas import tpu_sc as plsc

N = 131072
L = 16
F = 2
LOG2 = 19
TS = 2 ** LOG2
BASE_RES = 16
FINEST = int(BASE_RES * 2 ** (L - 1))
_B = np.exp((np.log(FINEST) - np.log(BASE_RES)) / (L - 1))
RES = [float(np.floor(BASE_RES * _B ** i)) for i in range(L)]
GS = [np.float32(1.0) / np.float32(r) for r in RES]
P1 = np.array(2654435761, np.uint32).astype(np.int32)
P2 = np.int32(805459861)

_col = np.arange(128)
_lvl = _col // 8
_cor = _col % 8
GS128 = np.array([GS[l] for l in _lvl], np.float32).reshape(1, 128)
CX128 = ((_cor >> 2) & 1).astype(np.int32).reshape(1, 128)
CY128 = ((_cor >> 1) & 1).astype(np.int32).reshape(1, 128)
CZ128 = (_cor & 1).astype(np.int32).reshape(1, 128)
OFF128 = (_lvl * TS).astype(np.int32).reshape(1, 128)
_col2 = np.arange(256)
_lvl2 = _col2 // 16
_cor2 = (_col2 % 16) // 2
GS256 = np.array([GS[l] for l in _lvl2], np.float32).reshape(1, 256)
CXF = ((_cor2 >> 2) & 1).astype(np.float32).reshape(1, 256)
CYF = ((_cor2 >> 1) & 1).astype(np.float32).reshape(1, 256)
CZF = (_cor2 & 1).astype(np.float32).reshape(1, 256)
ROWMAP = ((_col2 // 16) * 2 + (_col2 % 2)).astype(np.int32)

BA = 2048
BC = 1024


def _stage_a_body(pos, gs1, cx, cy, cz, off, gs2, cxf, cyf, czf, idx_out, w_out):
    px = pos[:, 0:1]
    py = pos[:, 1:2]
    pz = pos[:, 2:3]
    g = gs1[...]
    bx = jnp.floor(px / g).astype(jnp.int32) + cx[...]
    by = jnp.floor(py / g).astype(jnp.int32) + cy[...]
    bz = jnp.floor(pz / g).astype(jnp.int32) + cz[...]
    h = bx ^ (by * P1) ^ (bz * P2)
    idx_out[...] = (h & (TS - 1)) + off[...]
    g2 = gs2[...]
    one = jnp.float32(1.0)

    def corner_w(p, cb):
        vmin = jnp.floor(p / g2) * g2
        w = (p - vmin) / ((vmin + g2) - vmin)
        return cb * w + (one - cb) * (one - w)

    w_out[...] = (corner_w(px, cxf[...]) * corner_w(py, cyf[...])
                  * corner_w(pz, czf[...]))


def _stage_a(position, consts):
    grid = (N // BA,)
    row = lambda shape: pl.BlockSpec(shape, lambda i: (0, 0))
    return pl.pallas_call(
        _stage_a_body,
        grid=grid,
        in_specs=[
            pl.BlockSpec((BA, 3), lambda i: (i, 0)),
            row((1, 128)), row((1, 128)), row((1, 128)), row((1, 128)),
            row((1, 128)),
            row((1, 256)), row((1, 256)), row((1, 256)), row((1, 256)),
        ],
        out_specs=[
            pl.BlockSpec((BA, 128), lambda i: (i, 0)),
            pl.BlockSpec((BA, 256), lambda i: (i, 0)),
        ],
        out_shape=[
            jax.ShapeDtypeStruct((N, 128), jnp.int32),
            jax.ShapeDtypeStruct((N, 256), jnp.float32),
        ],
    )(position, *consts)


NC = 2
NS = 16
NW = NC * NS
PPW = N // NW
CHUNK = 128
KFIRE = 8


def _gather_body(tab_hbm, idx_hbm, out_hbm, idx_v, emb_v, sem_g):
    wid = lax.axis_index("s") * NC + lax.axis_index("c")
    base = wid * PPW

    def chunk_body(ci, carry):
        p0 = base + ci * CHUNK
        pltpu.sync_copy(idx_hbm.at[pl.ds(p0, CHUNK)], idx_v)

        def fire_drain(gi, c2):
            q0 = gi * KFIRE
            for q in range(KFIRE):
                pltpu.async_copy(tab_hbm.at[idx_v.at[q0 + q]],
                                 emb_v.at[q0 + q], sem_g)
            for q in range(KFIRE):
                pltpu.make_async_copy(tab_hbm.at[idx_v.at[q0 + q]],
                                      emb_v.at[q0 + q], sem_g).wait()
            return c2

        lax.fori_loop(0, CHUNK // KFIRE, fire_drain, 0)
        pltpu.sync_copy(emb_v, out_hbm.at[pl.ds(p0, CHUNK)])
        return carry

    lax.fori_loop(0, PPW // CHUNK, chunk_body, 0)


_gather = functools.partial(
    pl.kernel,
    out_type=jax.ShapeDtypeStruct((N, 128, F), jnp.float32),
    mesh=plsc.VectorSubcoreMesh(core_axis_name="c", subcore_axis_name="s",
                                num_cores=NC, num_subcores=NS),
    scratch_types=[
        pltpu.VMEM((CHUNK, 128), jnp.int32),
        pltpu.VMEM((CHUNK, 128, F), jnp.float32),
        pltpu.SemaphoreType.DMA,
    ],
)(_gather_body)


def _stage_c_body(emb, w2, drc, mw0, db0, dw1, db1, cw0a, cw0b, cb0,
                  cw1, cb1, cw2, cb2, out):
    f32 = jnp.float32
    t = emb[...] * w2[...]
    h0 = jnp.maximum(jnp.dot(t, mw0[...], preferred_element_type=f32)
                     + db0[...], 0.0)
    dens = jnp.dot(h0, dw1[...], preferred_element_type=f32) + db1[...]
    sigma = jnp.maximum(dens[:, 15:16], 0.0)
    yd = jnp.dot(dens, cw0a[...], preferred_element_type=f32) + cb0[...]

    x = drc[:, 0:1]
    y = drc[:, 1:2]
    z = drc[:, 2:3]
    x2 = x * x; y2 = y * y; z2 = z * z
    xy = x * y; xz = x * z; yz = y * z
    x4 = x2 * x2; y4 = y2 * y2
    c1 = 0.5 * np.sqrt(3.0 / np.pi)
    sub = 0.25 * np.sqrt(5.0 / np.pi)
    v1 = 0.25 * np.sqrt(15.0 / np.pi)
    v2 = 0.5 * np.sqrt(15.0 / np.pi)
    v3 = 0.75 * np.sqrt(5.0 / np.pi)
    w1c = 0.25 * np.sqrt(105.0 / np.pi)
    w2c = 0.5 * np.sqrt(105.0 / np.pi)
    w3c = 0.25 * np.sqrt(35.0 / (2.0 * np.pi))
    w4c = 0.5 * np.sqrt(7.0 / (6.0 * np.pi))
    ones = jnp.ones_like(x)
    basis = [
        0.5 * np.sqrt(1.0 / np.pi) * ones,
        -c1 * y, c1 * z, -c1 * x,
        v2 * xy, -v2 * yz, v3 * z2 - sub, -v2 * xz, v1 * x2 - v1 * y2,
        -w3c * y * (3.0 * x2 - y2),
        w2c * xy * z,
        w4c * y * (1.5 - 7.5 * z2),
        1.24392110863372 * z * (1.5 * z2 - 0.5) - 0.497568443453487 * z,
        w4c * x * (1.5 - 7.5 * z2),
        w1c * z * (x2 - y2),
        -w3c * x * (x2 - 3.0 * y2),
        2.5033429417967 * xy * (x2 - y2),
        -1.77013076977993 * yz * (3.0 * x2 - y2),
        0.126156626101008 * xy * (52.5 * z2 - 7.5),
        0.267618617422916 * y * (2.33333333333333 * z * (1.5 - 7.5 * z2) + 4.0 * z),
        1.48099765681286 * z * (1.66666666666667 * z * (1.5 * z2 - 0.5) - 0.666666666666667 * z) - 0.952069922236839 * z2 + 0.317356640745613,
        0.267618617422916 * x * (2.33333333333333 * z * (1.5 - 7.5 * z2) + 4.0 * z),
        0.063078313050504 * (x2 - y2) * (52.5 * z2 - 7.5),
        -1.77013076977993 * xz * (x2 - 3.0 * y2),
        -3.75501441269506 * x2 * y2 + 0.625835735449176 * x4 + 0.625835735449176 * y4,
    ]
    for i, b in enumerate(basis):
        yd = yd + b * cw0b[i:i + 1, :]
    h1 = jnp.maximum(yd, 0.0)
    h2 = jnp.maximum(jnp.dot(h1, cw1[...], preferred_element_type=f32)
                     + cb1[...], 0.0)
    rgb = jnp.dot(h2, cw2[...], preferred_element_type=f32) + cb2[...]
    out[...] = jnp.concatenate([rgb, sigma], axis=1)


def _stage_c(emb2, w2, direction, weights):
    grid = (N // BC,)

    def full(a):
        return pl.BlockSpec(a.shape, lambda i: tuple(0 for _ in a.shape))

    return pl.pallas_call(
        _stage_c_body,
        grid=grid,
        in_specs=[
            pl.BlockSpec((BC, 256), lambda i: (i, 0)),
            pl.BlockSpec((BC, 256), lambda i: (i, 0)),
            pl.BlockSpec((BC, 3), lambda i: (i, 0)),
        ] + [full(wt) for wt in weights],
        out_specs=pl.BlockSpec((BC, 4), lambda i: (i, 0)),
        out_shape=jax.ShapeDtypeStruct((N, 4), jnp.float32),
    )(emb2, w2, direction, *weights)


def kernel(position, direction, tables, dW0, db0, dW1, db1, cW0, cb0,
           cW1, cb1, cW2, cb2):
    consts = [jnp.asarray(a) for a in
              (GS128, CX128, CY128, CZ128, OFF128, GS256, CXF, CYF, CZF)]
    idx, w2 = _stage_a(position, consts)
    tab_flat = tables.reshape(L * TS, F)
    emb = _gather(tab_flat, idx)
    emb2 = emb.reshape(N, 256)
    weights = [
        jnp.take(dW0, jnp.asarray(ROWMAP), axis=0),
        db0.reshape(1, -1),
        dW1, db1.reshape(1, -1),
        cW0[:16], cW0[16:], cb0.reshape(1, -1),
        cW1, cb1.reshape(1, -1),
        cW2, cb2.reshape(1, -1),
    ]
    return _stage_c(emb2, w2, direction, weights)


# speedup vs baseline: 10.2660x; 10.2660x over previous
import functools

import numpy as np
import jax
import jax.numpy as jnp
from jax import lax
from jax.experimental import pallas as pl
from jax.experimental.pallas import tpu as pltpu
from jax.experimental.pallas import tpu_sc as plsc

N = 131072
L = 16
F = 2
LOG2 = 19
TS = 2 ** LOG2
BASE_RES = 16
FINEST = int(BASE_RES * 2 ** (L - 1))
_B = np.exp((np.log(FINEST) - np.log(BASE_RES)) / (L - 1))
RES = [float(np.floor(BASE_RES * _B ** i)) for i in range(L)]
GS = [np.float32(1.0) / np.float32(r) for r in RES]
P1 = np.array(2654435761, np.uint32).astype(np.int32)
P2 = np.int32(805459861)

_col = np.arange(128)
_lvl = _col // 8
_cor = _col % 8
GS128 = np.array([GS[l] for l in _lvl], np.float32).reshape(1, 128)
CX128 = ((_cor >> 2) & 1).astype(np.int32).reshape(1, 128)
CY128 = ((_cor >> 1) & 1).astype(np.int32).reshape(1, 128)
CZ128 = (_cor & 1).astype(np.int32).reshape(1, 128)
OFF128 = (_lvl * TS).astype(np.int32).reshape(1, 128)
_col2 = np.arange(256)
_lvl2 = _col2 // 16
_cor2 = (_col2 % 16) // 2
GS256 = np.array([GS[l] for l in _lvl2], np.float32).reshape(1, 256)
CXF = ((_cor2 >> 2) & 1).astype(np.float32).reshape(1, 256)
CYF = ((_cor2 >> 1) & 1).astype(np.float32).reshape(1, 256)
CZF = (_cor2 & 1).astype(np.float32).reshape(1, 256)
ROWMAP = ((_col2 // 16) * 2 + (_col2 % 2)).astype(np.int32)

BA = 2048
BC = 1024


def _stage_a_body(pos, gs1, cx, cy, cz, off, gs2, cxf, cyf, czf,
                  idx_out, lo_out, w_out):
    px = pos[:, 0:1]
    py = pos[:, 1:2]
    pz = pos[:, 2:3]
    g = gs1[...]
    bx = jnp.floor(px / g).astype(jnp.int32) + cx[...]
    by = jnp.floor(py / g).astype(jnp.int32) + cy[...]
    bz = jnp.floor(pz / g).astype(jnp.int32) + cz[...]
    h = bx ^ (by * P1) ^ (bz * P2)
    flat = (h & (TS - 1)) + off[...]
    idx_out[...] = flat >> 2
    lo_out[...] = (flat & 3) * 2
    g2 = gs2[...]
    one = jnp.float32(1.0)

    def corner_w(p, cb):
        vmin = jnp.floor(p / g2) * g2
        w = (p - vmin) / ((vmin + g2) - vmin)
        return cb * w + (one - cb) * (one - w)

    w_out[...] = (corner_w(px, cxf[...]) * corner_w(py, cyf[...])
                  * corner_w(pz, czf[...]))


def _stage_a(position, consts):
    grid = (N // BA,)
    row = lambda shape: pl.BlockSpec(shape, lambda i: (0, 0))
    return pl.pallas_call(
        _stage_a_body,
        grid=grid,
        in_specs=[
            pl.BlockSpec((BA, 3), lambda i: (i, 0)),
            row((1, 128)), row((1, 128)), row((1, 128)), row((1, 128)),
            row((1, 128)),
            row((1, 256)), row((1, 256)), row((1, 256)), row((1, 256)),
        ],
        out_specs=[
            pl.BlockSpec((BA, 128), lambda i: (i, 0)),
            pl.BlockSpec((BA, 128), lambda i: (i, 0)),
            pl.BlockSpec((BA, 256), lambda i: (i, 0)),
        ],
        out_shape=[
            jax.ShapeDtypeStruct((N, 128), jnp.int32),
            jax.ShapeDtypeStruct((N, 128), jnp.int32),
            jax.ShapeDtypeStruct((N, 256), jnp.float32),
        ],
    )(position, *consts)


NC = 2
NS = 16
NW = NC * NS
PPW = N // NW
CHUNK = 64
KFIRE = 8
LANES = 16


def _gather_body(tab_hbm, idx_hbm, lo_hbm, w_hbm, out_hbm,
                 idx_v, lo_v, w_v, buf_v, out_v, sem_g):
    wid = lax.axis_index("s") * NC + lax.axis_index("c")
    base = wid * PPW
    lane = lax.iota(jnp.int32, LANES)

    def chunk_body(ci, carry):
        p0 = base + ci * CHUNK
        pltpu.sync_copy(idx_hbm.at[pl.ds(p0, CHUNK)], idx_v)
        pltpu.sync_copy(lo_hbm.at[pl.ds(p0, CHUNK)], lo_v)
        pltpu.sync_copy(w_hbm.at[pl.ds(p0, CHUNK)], w_v)

        def fire_drain(gi, c2):
            q0 = gi * KFIRE
            for q in range(KFIRE):
                pltpu.async_copy(tab_hbm.at[idx_v.at[q0 + q]],
                                 buf_v.at[q0 + q], sem_g)
            for q in range(KFIRE):
                pltpu.make_async_copy(tab_hbm.at[idx_v.at[q0 + q]],
                                      buf_v.at[q0 + q], sem_g).wait()
            return c2

        lax.fori_loop(0, CHUNK // KFIRE, fire_drain, 0)

        def group_body(g, c3):
            pvec = g * LANES + lane

            def level_body(l, c4):
                acc0 = jnp.zeros((LANES,), jnp.float32)
                acc1 = jnp.zeros((LANES,), jnp.float32)
                for c in range(8):
                    j = l * 8 + c
                    jv = jnp.full((LANES,), j, jnp.int32)
                    s = plsc.load_gather(lo_v, [pvec, jv])
                    e0 = plsc.load_gather(buf_v, [pvec, jv, s])
                    e1 = plsc.load_gather(buf_v, [pvec, jv, s + 1])
                    w0 = plsc.load_gather(w_v, [pvec, jv * 2])
                    w1 = plsc.load_gather(w_v, [pvec, jv * 2 + 1])
                    acc0 = acc0 + w0 * e0
                    acc1 = acc1 + w1 * e1
                plsc.store_scatter(out_v, [pvec, jnp.full((LANES,), 0, jnp.int32) + l * 2], acc0)
                plsc.store_scatter(out_v, [pvec, jnp.full((LANES,), 1, jnp.int32) + l * 2], acc1)
                return c4

            lax.fori_loop(0, L, level_body, 0)
            return c3

        lax.fori_loop(0, CHUNK // LANES, group_body, 0)
        pltpu.sync_copy(out_v, out_hbm.at[pl.ds(p0, CHUNK)])
        return carry

    lax.fori_loop(0, PPW // CHUNK, chunk_body, 0)


@functools.cache
def _make_gather():
    return pl.kernel(
        _gather_body,
        out_type=jax.ShapeDtypeStruct((N, 2 * L), jnp.float32),
        mesh=plsc.VectorSubcoreMesh(core_axis_name="c", subcore_axis_name="s",
                                    num_cores=NC, num_subcores=NS),
        scratch_types=[
            pltpu.VMEM((CHUNK, 128), jnp.int32),
            pltpu.VMEM((CHUNK, 128), jnp.int32),
            pltpu.VMEM((CHUNK, 256), jnp.float32),
            pltpu.VMEM((CHUNK, 128, 8), jnp.float32),
            pltpu.VMEM((CHUNK, 2 * L), jnp.float32),
            pltpu.SemaphoreType.DMA,
        ],
        compiler_params=pltpu.CompilerParams(use_tc_tiling_on_sc=False,
                                             needs_layout_passes=False),
    )


def _stage_c_body(enc, drc, dw0, db0, dw1, db1, cw0a, cw0b, cb0,
                  cw1, cb1, cw2, cb2, out):
    f32 = jnp.float32
    h0 = jnp.maximum(jnp.dot(enc[...], dw0[...], preferred_element_type=f32)
                     + db0[...], 0.0)
    dens = jnp.dot(h0, dw1[...], preferred_element_type=f32) + db1[...]
    sigma = jnp.maximum(dens[:, 15:16], 0.0)
    yd = jnp.dot(dens, cw0a[...], preferred_element_type=f32) + cb0[...]

    x = drc[:, 0:1]
    y = drc[:, 1:2]
    z = drc[:, 2:3]
    x2 = x * x; y2 = y * y; z2 = z * z
    xy = x * y; xz = x * z; yz = y * z
    x4 = x2 * x2; y4 = y2 * y2
    c1 = 0.5 * np.sqrt(3.0 / np.pi)
    sub = 0.25 * np.sqrt(5.0 / np.pi)
    v1 = 0.25 * np.sqrt(15.0 / np.pi)
    v2 = 0.5 * np.sqrt(15.0 / np.pi)
    v3 = 0.75 * np.sqrt(5.0 / np.pi)
    w1c = 0.25 * np.sqrt(105.0 / np.pi)
    w2c = 0.5 * np.sqrt(105.0 / np.pi)
    w3c = 0.25 * np.sqrt(35.0 / (2.0 * np.pi))
    w4c = 0.5 * np.sqrt(7.0 / (6.0 * np.pi))
    ones = jnp.ones_like(x)
    basis = [
        0.5 * np.sqrt(1.0 / np.pi) * ones,
        -c1 * y, c1 * z, -c1 * x,
        v2 * xy, -v2 * yz, v3 * z2 - sub, -v2 * xz, v1 * x2 - v1 * y2,
        -w3c * y * (3.0 * x2 - y2),
        w2c * xy * z,
        w4c * y * (1.5 - 7.5 * z2),
        1.24392110863372 * z * (1.5 * z2 - 0.5) - 0.497568443453487 * z,
        w4c * x * (1.5 - 7.5 * z2),
        w1c * z * (x2 - y2),
        -w3c * x * (x2 - 3.0 * y2),
        2.5033429417967 * xy * (x2 - y2),
        -1.77013076977993 * yz * (3.0 * x2 - y2),
        0.126156626101008 * xy * (52.5 * z2 - 7.5),
        0.267618617422916 * y * (2.33333333333333 * z * (1.5 - 7.5 * z2) + 4.0 * z),
        1.48099765681286 * z * (1.66666666666667 * z * (1.5 * z2 - 0.5) - 0.666666666666667 * z) - 0.952069922236839 * z2 + 0.317356640745613,
        0.267618617422916 * x * (2.33333333333333 * z * (1.5 - 7.5 * z2) + 4.0 * z),
        0.063078313050504 * (x2 - y2) * (52.5 * z2 - 7.5),
        -1.77013076977993 * xz * (x2 - 3.0 * y2),
        -3.75501441269506 * x2 * y2 + 0.625835735449176 * x4 + 0.625835735449176 * y4,
    ]
    for i, b in enumerate(basis):
        yd = yd + b * cw0b[i:i + 1, :]
    h1 = jnp.maximum(yd, 0.0)
    h2 = jnp.maximum(jnp.dot(h1, cw1[...], preferred_element_type=f32)
                     + cb1[...], 0.0)
    rgb = jnp.dot(h2, cw2[...], preferred_element_type=f32) + cb2[...]
    out[...] = jnp.concatenate([rgb, sigma], axis=1)


def _stage_c(enc, direction, weights):
    grid = (N // BC,)

    def full(a):
        return pl.BlockSpec(a.shape, lambda i: tuple(0 for _ in a.shape))

    return pl.pallas_call(
        _stage_c_body,
        grid=grid,
        in_specs=[
            pl.BlockSpec((BC, 2 * L), lambda i: (i, 0)),
            pl.BlockSpec((BC, 3), lambda i: (i, 0)),
        ] + [full(wt) for wt in weights],
        out_specs=pl.BlockSpec((BC, 4), lambda i: (i, 0)),
        out_shape=jax.ShapeDtypeStruct((N, 4), jnp.float32),
    )(enc, direction, *weights)


def kernel(position, direction, tables, dW0, db0, dW1, db1, cW0, cb0,
           cW1, cb1, cW2, cb2):
    consts = [jnp.asarray(a) for a in
              (GS128, CX128, CY128, CZ128, OFF128, GS256, CXF, CYF, CZF)]
    idx, lo2, w2 = _stage_a(position, consts)
    tab8 = tables.reshape(L * TS // 4, 8)
    enc = _make_gather()(tab8, idx, lo2, w2)
    weights = [
        dW0, db0.reshape(1, -1),
        dW1, db1.reshape(1, -1),
        cW0[:16], cW0[16:], cb0.reshape(1, -1),
        cW1, cb1.reshape(1, -1),
        cW2, cb2.reshape(1, -1),
    ]
    return _stage_c(enc, direction, weights)


# speedup vs baseline: 10.3079x; 1.0041x over previous
import functools

import numpy as np
import jax
import jax.numpy as jnp
from jax import lax
from jax.experimental import pallas as pl
from jax.experimental.pallas import tpu as pltpu
from jax.experimental.pallas import tpu_sc as plsc

N = 131072
L = 16
F = 2
LOG2 = 19
TS = 2 ** LOG2
BASE_RES = 16
FINEST = int(BASE_RES * 2 ** (L - 1))
_B = np.exp((np.log(FINEST) - np.log(BASE_RES)) / (L - 1))
RES = [float(np.floor(BASE_RES * _B ** i)) for i in range(L)]
GSF = [float(np.float32(1.0) / np.float32(r)) for r in RES]
P1 = int(np.array(2654435761, np.uint32).astype(np.int32))
P2 = 805459861

BC = 1024

NC = 2
NS = 16
NW = NC * NS
PPW = N // NW
CHUNK = 64
KFIRE = 8
LANES = 16


def _enc_body(tab_hbm, pos_hbm, out_hbm, pos_v, idx_v, lo_v, buf_v, out_v,
              sem_g):
    wid = lax.axis_index("s") * NC + lax.axis_index("c")
    base = wid * PPW
    lane = lax.iota(jnp.int32, LANES)

    def cvec(v):
        return jnp.full((LANES,), v, jnp.int32)

    def load_xyz(pvec):
        x = plsc.load_gather(pos_v, [pvec, cvec(0)])
        y = plsc.load_gather(pos_v, [pvec, cvec(1)])
        z = plsc.load_gather(pos_v, [pvec, cvec(2)])
        return x, y, z

    def chunk_body(ci, carry):
        p0 = base + ci * CHUNK
        pltpu.sync_copy(pos_hbm.at[pl.ds(p0, CHUNK)], pos_v)

        def hash_group(g, c2):
            pvec = g * LANES + lane
            x, y, z = load_xyz(pvec)
            for l in range(L):
                res = np.float32(RES[l])
                bx = (x * res).astype(jnp.int32)
                by = (y * res).astype(jnp.int32)
                bz = (z * res).astype(jnp.int32)
                hx = (bx, bx + 1)
                hy = (by * P1, (by + 1) * P1)
                hz = (bz * P2, (bz + 1) * P2)
                for c in range(8):
                    ib, jb, kb = (c >> 2) & 1, (c >> 1) & 1, c & 1
                    flat = ((hx[ib] ^ hy[jb] ^ hz[kb]) & (TS - 1)) + l * TS
                    col = cvec(l * 8 + c)
                    plsc.store_scatter(idx_v, [pvec, col], flat >> 2)
                    plsc.store_scatter(lo_v, [pvec, col], (flat & 3) * 2)
            return c2

        lax.fori_loop(0, CHUNK // LANES, hash_group, 0)

        def fire_drain(gi, c2):
            q0 = gi * KFIRE
            for q in range(KFIRE):
                pltpu.async_copy(tab_hbm.at[idx_v.at[q0 + q]],
                                 buf_v.at[q0 + q], sem_g)
            for q in range(KFIRE):
                pltpu.make_async_copy(tab_hbm.at[idx_v.at[q0 + q]],
                                      buf_v.at[q0 + q], sem_g).wait()
            return c2

        lax.fori_loop(0, CHUNK // KFIRE, fire_drain, 0)

        def group_body(g, c3):
            pvec = g * LANES + lane
            x, y, z = load_xyz(pvec)
            for l in range(L):
                res = np.float32(RES[l])
                gs = np.float32(GSF[l])

                def frac(p):
                    b = (p * res).astype(jnp.int32).astype(jnp.float32)
                    return (p - b * gs) * res

                wx, wy, wz = frac(x), frac(y), frac(z)
                sx = (1.0 - wx, wx)
                sy = (1.0 - wy, wy)
                sz = (1.0 - wz, wz)
                acc0 = jnp.zeros((LANES,), jnp.float32)
                acc1 = jnp.zeros((LANES,), jnp.float32)
                for c in range(8):
                    ib, jb, kb = (c >> 2) & 1, (c >> 1) & 1, c & 1
                    wk = sx[ib] * sy[jb] * sz[kb]
                    col = cvec(l * 8 + c)
                    s = plsc.load_gather(lo_v, [pvec, col])
                    e0 = plsc.load_gather(buf_v, [pvec, col, s])
                    e1 = plsc.load_gather(buf_v, [pvec, col, s + 1])
                    acc0 = acc0 + wk * e0
                    acc1 = acc1 + wk * e1
                plsc.store_scatter(out_v, [pvec, cvec(l * 2)], acc0)
                plsc.store_scatter(out_v, [pvec, cvec(l * 2 + 1)], acc1)
            return c3

        lax.fori_loop(0, CHUNK // LANES, group_body, 0)
        pltpu.sync_copy(out_v, out_hbm.at[pl.ds(p0, CHUNK)])
        return carry

    lax.fori_loop(0, PPW // CHUNK, chunk_body, 0)


@functools.cache
def _make_enc():
    return pl.kernel(
        _enc_body,
        out_type=jax.ShapeDtypeStruct((N, 2 * L), jnp.float32),
        mesh=plsc.VectorSubcoreMesh(core_axis_name="c", subcore_axis_name="s",
                                    num_cores=NC, num_subcores=NS),
        scratch_types=[
            pltpu.VMEM((CHUNK, 3), jnp.float32),
            pltpu.VMEM((CHUNK, 128), jnp.int32),
            pltpu.VMEM((CHUNK, 128), jnp.int32),
            pltpu.VMEM((CHUNK, 128, 8), jnp.float32),
            pltpu.VMEM((CHUNK, 2 * L), jnp.float32),
            pltpu.SemaphoreType.DMA,
        ],
        compiler_params=pltpu.CompilerParams(use_tc_tiling_on_sc=False,
                                             needs_layout_passes=False),
    )


def _stage_c_body(enc, drc, dw0, db0, dw1, db1, cw0a, cw0b, cb0,
                  cw1, cb1, cw2, cb2, out):
    f32 = jnp.float32
    h0 = jnp.maximum(jnp.dot(enc[...], dw0[...], preferred_element_type=f32)
                     + db0[...], 0.0)
    dens = jnp.dot(h0, dw1[...], preferred_element_type=f32) + db1[...]
    sigma = jnp.maximum(dens[:, 15:16], 0.0)
    yd = jnp.dot(dens, cw0a[...], preferred_element_type=f32) + cb0[...]

    x = drc[:, 0:1]
    y = drc[:, 1:2]
    z = drc[:, 2:3]
    x2 = x * x; y2 = y * y; z2 = z * z
    xy = x * y; xz = x * z; yz = y * z
    x4 = x2 * x2; y4 = y2 * y2
    c1 = 0.5 * np.sqrt(3.0 / np.pi)
    sub = 0.25 * np.sqrt(5.0 / np.pi)
    v1 = 0.25 * np.sqrt(15.0 / np.pi)
    v2 = 0.5 * np.sqrt(15.0 / np.pi)
    v3 = 0.75 * np.sqrt(5.0 / np.pi)
    w1c = 0.25 * np.sqrt(105.0 / np.pi)
    w2c = 0.5 * np.sqrt(105.0 / np.pi)
    w3c = 0.25 * np.sqrt(35.0 / (2.0 * np.pi))
    w4c = 0.5 * np.sqrt(7.0 / (6.0 * np.pi))
    ones = jnp.ones_like(x)
    basis = [
        0.5 * np.sqrt(1.0 / np.pi) * ones,
        -c1 * y, c1 * z, -c1 * x,
        v2 * xy, -v2 * yz, v3 * z2 - sub, -v2 * xz, v1 * x2 - v1 * y2,
        -w3c * y * (3.0 * x2 - y2),
        w2c * xy * z,
        w4c * y * (1.5 - 7.5 * z2),
        1.24392110863372 * z * (1.5 * z2 - 0.5) - 0.497568443453487 * z,
        w4c * x * (1.5 - 7.5 * z2),
        w1c * z * (x2 - y2),
        -w3c * x * (x2 - 3.0 * y2),
        2.5033429417967 * xy * (x2 - y2),
        -1.77013076977993 * yz * (3.0 * x2 - y2),
        0.126156626101008 * xy * (52.5 * z2 - 7.5),
        0.267618617422916 * y * (2.33333333333333 * z * (1.5 - 7.5 * z2) + 4.0 * z),
        1.48099765681286 * z * (1.66666666666667 * z * (1.5 * z2 - 0.5) - 0.666666666666667 * z) - 0.952069922236839 * z2 + 0.317356640745613,
        0.267618617422916 * x * (2.33333333333333 * z * (1.5 - 7.5 * z2) + 4.0 * z),
        0.063078313050504 * (x2 - y2) * (52.5 * z2 - 7.5),
        -1.77013076977993 * xz * (x2 - 3.0 * y2),
        -3.75501441269506 * x2 * y2 + 0.625835735449176 * x4 + 0.625835735449176 * y4,
    ]
    for i, b in enumerate(basis):
        yd = yd + b * cw0b[i:i + 1, :]
    h1 = jnp.maximum(yd, 0.0)
    h2 = jnp.maximum(jnp.dot(h1, cw1[...], preferred_element_type=f32)
                     + cb1[...], 0.0)
    rgb = jnp.dot(h2, cw2[...], preferred_element_type=f32) + cb2[...]
    out[...] = jnp.concatenate([rgb, sigma], axis=1)


def _stage_c(enc, direction, weights):
    grid = (N // BC,)

    def full(a):
        return pl.BlockSpec(a.shape, lambda i: tuple(0 for _ in a.shape))

    return pl.pallas_call(
        _stage_c_body,
        grid=grid,
        in_specs=[
            pl.BlockSpec((BC, 2 * L), lambda i: (i, 0)),
            pl.BlockSpec((BC, 3), lambda i: (i, 0)),
        ] + [full(wt) for wt in weights],
        out_specs=pl.BlockSpec((BC, 4), lambda i: (i, 0)),
        out_shape=jax.ShapeDtypeStruct((N, 4), jnp.float32),
    )(enc, direction, *weights)


def kernel(position, direction, tables, dW0, db0, dW1, db1, cW0, cb0,
           cW1, cb1, cW2, cb2):
    tab8 = tables.reshape(L * TS // 4, 8)
    enc = _make_enc()(tab8, position)
    weights = [
        dW0, db0.reshape(1, -1),
        dW1, db1.reshape(1, -1),
        cW0[:16], cW0[16:], cb0.reshape(1, -1),
        cW1, cb1.reshape(1, -1),
        cW2, cb2.reshape(1, -1),
    ]
    return _stage_c(enc, direction, weights)
